# baseline (speedup 1.0000x reference)
import functools

import jax
import jax.numpy as jnp
from jax import lax
from jax.experimental import pallas as pl
from jax.experimental.pallas import tpu as pltpu
from jax.experimental.pallas import tpu_sc as plsc

_NC, _NS = 2, 16
_NW = _NC * _NS
_CHUNK = 16
_NBUF = 4
_LAG = 2
_REP = 1024


def _sc_gather(table, idx_flat, s_dim, n_dim):
    B = idx_flat.shape[0]
    D = table.shape[1]
    b_per_w = B // _NW
    chunks = b_per_w // _CHUNK
    assert chunks % _NBUF == 0 and chunks >= 2 * _NBUF
    groups = chunks // _NBUF
    mesh = plsc.VectorSubcoreMesh(core_axis_name="c", subcore_axis_name="s")

    @functools.partial(
        pl.kernel,
        mesh=mesh,
        out_type=jax.ShapeDtypeStruct((s_dim, n_dim, D), jnp.float32),
        scratch_types=[
            pltpu.VMEM((b_per_w,), jnp.int32),
        ]
        + [pltpu.VMEM((_CHUNK, D), jnp.float32)] * _NBUF
        + [pltpu.SemaphoreType.DMA] * (2 * _NBUF),
    )
    def k(table_hbm, idx_hbm, out3d_hbm, idx_v, *bufs_and_sems):
        rows = bufs_and_sems[:_NBUF]
        gsem = bufs_and_sems[_NBUF : 2 * _NBUF]
        wsem = bufs_and_sems[2 * _NBUF :]

        out_hbm = out3d_hbm.reshape(B, D)
        wid = lax.axis_index("s") * _NC + lax.axis_index("c")
        base = wid * b_per_w
        pltpu.sync_copy(idx_hbm.at[pl.ds(base, b_per_w)], idx_v)

        def start_gather(c, b):
            pltpu.make_async_copy(
                table_hbm.at[idx_v.at[pl.ds(c * _CHUNK, _CHUNK)]],
                rows[b],
                gsem[b],
            ).start()

        def wait_gather(b):
            pltpu.make_async_copy(
                table_hbm.at[idx_v.at[pl.ds(0, _CHUNK)]], rows[b], gsem[b]
            ).wait()

        def start_write(c, b):
            pltpu.make_async_copy(
                rows[b], out_hbm.at[pl.ds(base + c * _CHUNK, _CHUNK)], wsem[b]
            ).start()

        def wait_write(b):
            pltpu.make_async_copy(
                rows[b], out_hbm.at[pl.ds(base, _CHUNK)], wsem[b]
            ).wait()

        for b in range(_NBUF):
            start_gather(b, b)
        for j in range(_NBUF - _LAG):
            wait_gather(j % _NBUF)
            start_write(j, j % _NBUF)

        @pl.loop(1, groups)
        def _(g):
            c_base = g * _NBUF
            for b in range(_NBUF):
                i = c_base + b
                bj = (b - _LAG) % _NBUF
                wait_write(b)
                start_gather(i, b)
                wait_gather(bj)
                start_write(i - _LAG, bj)

        for j in range(chunks - _LAG, chunks):
            bj = j % _NBUF
            wait_gather(bj)
            start_write(j, bj)
        for b in range(_NBUF):
            wait_write(b)

    return k(table, idx_flat)


def kernel(seq_input, token_type_input, token_type_embeddings):
    s, n = seq_input.shape
    if token_type_input is None:
        token_type_input = jnp.zeros((s, n), dtype=jnp.int32)
    B = s * n
    idx_flat = token_type_input.reshape(-1)
    if _REP > 1:
        table = jnp.tile(token_type_embeddings, (_REP, 1))
        idx_flat = idx_flat + 2 * (jnp.arange(B, dtype=jnp.int32) % _REP)
    else:
        table = token_type_embeddings
    return _sc_gather(table, idx_flat, s, n)


# speedup vs baseline: 1.1547x; 1.1547x over previous
import functools

import jax
import jax.numpy as jnp
from jax import lax
from jax.experimental import pallas as pl
from jax.experimental.pallas import tpu as pltpu
from jax.experimental.pallas import tpu_sc as plsc

_NC, _NS = 2, 16
_NW = _NC * _NS
_CHUNK = 16
_NBUF = 4
_LAG = 2
_REP = 896


def _sc_gather(table, idx_flat, s_dim, n_dim):
    B = idx_flat.shape[0]
    D = table.shape[1]
    b_per_w = B // _NW
    chunks = b_per_w // _CHUNK
    assert chunks % _NBUF == 0 and chunks >= 2 * _NBUF
    groups = chunks // _NBUF
    mesh = plsc.VectorSubcoreMesh(core_axis_name="c", subcore_axis_name="s")

    @functools.partial(
        pl.kernel,
        mesh=mesh,
        out_type=jax.ShapeDtypeStruct((s_dim, n_dim, D), jnp.float32),
        scratch_types=[
            pltpu.VMEM((b_per_w,), jnp.int32),
        ]
        + [pltpu.VMEM((_CHUNK, D), jnp.float32)] * _NBUF
        + [pltpu.SemaphoreType.DMA] * (2 * _NBUF),
    )
    def k(table_hbm, idx_hbm, out3d_hbm, idx_v, *bufs_and_sems):
        rows = bufs_and_sems[:_NBUF]
        gsem = bufs_and_sems[_NBUF : 2 * _NBUF]
        wsem = bufs_and_sems[2 * _NBUF :]

        out_hbm = out3d_hbm.reshape(B, D)
        wid = lax.axis_index("s") * _NC + lax.axis_index("c")
        base = wid * b_per_w
        pltpu.sync_copy(idx_hbm.at[pl.ds(base, b_per_w)], idx_v)

        def start_gather(c, b):
            pltpu.make_async_copy(
                table_hbm.at[idx_v.at[pl.ds(c * _CHUNK, _CHUNK)]],
                rows[b],
                gsem[b],
            ).start()

        def wait_gather(b):
            pltpu.make_async_copy(
                table_hbm.at[idx_v.at[pl.ds(0, _CHUNK)]], rows[b], gsem[b]
            ).wait()

        def start_write(c, b):
            pltpu.make_async_copy(
                rows[b], out_hbm.at[pl.ds(base + c * _CHUNK, _CHUNK)], wsem[b]
            ).start()

        def wait_write(b):
            pltpu.make_async_copy(
                rows[b], out_hbm.at[pl.ds(base, _CHUNK)], wsem[b]
            ).wait()

        for b in range(_NBUF):
            start_gather(b, b)
        for j in range(_NBUF - _LAG):
            wait_gather(j % _NBUF)
            start_write(j, j % _NBUF)

        @pl.loop(1, groups)
        def _(g):
            c_base = g * _NBUF
            for b in range(_NBUF):
                i = c_base + b
                bj = (b - _LAG) % _NBUF
                wait_write(b)
                start_gather(i, b)
                wait_gather(bj)
                start_write(i - _LAG, bj)

        for j in range(chunks - _LAG, chunks):
            bj = j % _NBUF
            wait_gather(bj)
            start_write(j, bj)
        for b in range(_NBUF):
            wait_write(b)

    return k(table, idx_flat)


def kernel(seq_input, token_type_input, token_type_embeddings):
    s, n = seq_input.shape
    if token_type_input is None:
        token_type_input = jnp.zeros((s, n), dtype=jnp.int32)
    B = s * n
    idx_flat = token_type_input.reshape(-1)
    if _REP > 1:
        table = jnp.tile(token_type_embeddings, (_REP, 1))
        idx_flat = idx_flat + 2 * (jnp.arange(B, dtype=jnp.int32) % _REP)
    else:
        table = token_type_embeddings
    return _sc_gather(table, idx_flat, s, n)


# speedup vs baseline: 1.1590x; 1.0037x over previous
import functools

import jax
import jax.numpy as jnp
from jax import lax
from jax.experimental import pallas as pl
from jax.experimental.pallas import tpu as pltpu
from jax.experimental.pallas import tpu_sc as plsc

_NC, _NS = 2, 16
_NW = _NC * _NS
_CHUNK = 16
_NBUF = 4
_LAG = 2
_REP = 960


def _sc_gather(table, idx_flat, s_dim, n_dim):
    B = idx_flat.shape[0]
    D = table.shape[1]
    b_per_w = B // _NW
    chunks = b_per_w // _CHUNK
    assert chunks % _NBUF == 0 and chunks >= 2 * _NBUF
    groups = chunks // _NBUF
    mesh = plsc.VectorSubcoreMesh(core_axis_name="c", subcore_axis_name="s")

    @functools.partial(
        pl.kernel,
        mesh=mesh,
        out_type=jax.ShapeDtypeStruct((s_dim, n_dim, D), jnp.float32),
        scratch_types=[
            pltpu.VMEM((b_per_w,), jnp.int32),
        ]
        + [pltpu.VMEM((_CHUNK, D), jnp.float32)] * _NBUF
        + [pltpu.SemaphoreType.DMA] * (2 * _NBUF),
    )
    def k(table_hbm, idx_hbm, out3d_hbm, idx_v, *bufs_and_sems):
        rows = bufs_and_sems[:_NBUF]
        gsem = bufs_and_sems[_NBUF : 2 * _NBUF]
        wsem = bufs_and_sems[2 * _NBUF :]

        out_hbm = out3d_hbm.reshape(B, D)
        wid = lax.axis_index("s") * _NC + lax.axis_index("c")
        base = wid * b_per_w
        pltpu.sync_copy(idx_hbm.at[pl.ds(base, b_per_w)], idx_v)

        def start_gather(c, b):
            pltpu.make_async_copy(
                table_hbm.at[idx_v.at[pl.ds(c * _CHUNK, _CHUNK)]],
                rows[b],
                gsem[b],
            ).start()

        def wait_gather(b):
            pltpu.make_async_copy(
                table_hbm.at[idx_v.at[pl.ds(0, _CHUNK)]], rows[b], gsem[b]
            ).wait()

        def start_write(c, b):
            pltpu.make_async_copy(
                rows[b], out_hbm.at[pl.ds(base + c * _CHUNK, _CHUNK)], wsem[b]
            ).start()

        def wait_write(b):
            pltpu.make_async_copy(
                rows[b], out_hbm.at[pl.ds(base, _CHUNK)], wsem[b]
            ).wait()

        for b in range(_NBUF):
            start_gather(b, b)
        for j in range(_NBUF - _LAG):
            wait_gather(j % _NBUF)
            start_write(j, j % _NBUF)

        @pl.loop(1, groups)
        def _(g):
            c_base = g * _NBUF
            for b in range(_NBUF):
                i = c_base + b
                bj = (b - _LAG) % _NBUF
                wait_write(b)
                start_gather(i, b)
                wait_gather(bj)
                start_write(i - _LAG, bj)

        for j in range(chunks - _LAG, chunks):
            bj = j % _NBUF
            wait_gather(bj)
            start_write(j, bj)
        for b in range(_NBUF):
            wait_write(b)

    return k(table, idx_flat)


def kernel(seq_input, token_type_input, token_type_embeddings):
    s, n = seq_input.shape
    if token_type_input is None:
        token_type_input = jnp.zeros((s, n), dtype=jnp.int32)
    B = s * n
    idx_flat = token_type_input.reshape(-1)
    if _REP > 1:
        table = jnp.tile(token_type_embeddings, (_REP, 1))
        idx_flat = idx_flat + 2 * (jnp.arange(B, dtype=jnp.int32) % _REP)
    else:
        table = token_type_embeddings
    return _sc_gather(table, idx_flat, s, n)


# speedup vs baseline: 1.1623x; 1.0028x over previous
import functools

import jax
import jax.numpy as jnp
from jax import lax
from jax.experimental import pallas as pl
from jax.experimental.pallas import tpu as pltpu
from jax.experimental.pallas import tpu_sc as plsc

_NC, _NS = 2, 16
_NW = _NC * _NS
_CHUNK = 16
_NBUF = 4
_LAG = 3
_REP = 960


def _sc_gather(table, idx_flat, s_dim, n_dim):
    B = idx_flat.shape[0]
    D = table.shape[1]
    b_per_w = B // _NW
    chunks = b_per_w // _CHUNK
    assert chunks % _NBUF == 0 and chunks >= 2 * _NBUF
    groups = chunks // _NBUF
    mesh = plsc.VectorSubcoreMesh(core_axis_name="c", subcore_axis_name="s")

    @functools.partial(
        pl.kernel,
        mesh=mesh,
        out_type=jax.ShapeDtypeStruct((s_dim, n_dim, D), jnp.float32),
        scratch_types=[
            pltpu.VMEM((b_per_w,), jnp.int32),
        ]
        + [pltpu.VMEM((_CHUNK, D), jnp.float32)] * _NBUF
        + [pltpu.SemaphoreType.DMA] * (2 * _NBUF),
    )
    def k(table_hbm, idx_hbm, out3d_hbm, idx_v, *bufs_and_sems):
        rows = bufs_and_sems[:_NBUF]
        gsem = bufs_and_sems[_NBUF : 2 * _NBUF]
        wsem = bufs_and_sems[2 * _NBUF :]

        out_hbm = out3d_hbm.reshape(B, D)
        wid = lax.axis_index("s") * _NC + lax.axis_index("c")
        base = wid * b_per_w
        pltpu.sync_copy(idx_hbm.at[pl.ds(base, b_per_w)], idx_v)

        def start_gather(c, b):
            pltpu.make_async_copy(
                table_hbm.at[idx_v.at[pl.ds(c * _CHUNK, _CHUNK)]],
                rows[b],
                gsem[b],
            ).start()

        def wait_gather(b):
            pltpu.make_async_copy(
                table_hbm.at[idx_v.at[pl.ds(0, _CHUNK)]], rows[b], gsem[b]
            ).wait()

        def start_write(c, b):
            pltpu.make_async_copy(
                rows[b], out_hbm.at[pl.ds(base + c * _CHUNK, _CHUNK)], wsem[b]
            ).start()

        def wait_write(b):
            pltpu.make_async_copy(
                rows[b], out_hbm.at[pl.ds(base, _CHUNK)], wsem[b]
            ).wait()

        for b in range(_NBUF):
            start_gather(b, b)
        for j in range(_NBUF - _LAG):
            wait_gather(j % _NBUF)
            start_write(j, j % _NBUF)

        @pl.loop(1, groups)
        def _(g):
            c_base = g * _NBUF
            for b in range(_NBUF):
                i = c_base + b
                bj = (b - _LAG) % _NBUF
                wait_write(b)
                start_gather(i, b)
                wait_gather(bj)
                start_write(i - _LAG, bj)

        for j in range(chunks - _LAG, chunks):
            bj = j % _NBUF
            wait_gather(bj)
            start_write(j, bj)
        for b in range(_NBUF):
            wait_write(b)

    return k(table, idx_flat)


def kernel(seq_input, token_type_input, token_type_embeddings):
    s, n = seq_input.shape
    if token_type_input is None:
        token_type_input = jnp.zeros((s, n), dtype=jnp.int32)
    B = s * n
    idx_flat = token_type_input.reshape(-1)
    if _REP > 1:
        table = jnp.tile(token_type_embeddings, (_REP, 1))
        idx_flat = idx_flat + 2 * (jnp.arange(B, dtype=jnp.int32) % _REP)
    else:
        table = token_type_embeddings
    return _sc_gather(table, idx_flat, s, n)
